# baseline (speedup 1.0000x reference)
import functools

import jax
import jax.numpy as jnp
from jax import lax
from jax.experimental import pallas as pl
from jax.experimental.pallas import tpu as pltpu
from jax.experimental.pallas import tpu_sc as plsc

NC, NS = 2, 16
NW = NC * NS


@functools.lru_cache(maxsize=None)
def _sc_gather(N, D, C):
    b_per_w = N // NW
    n_chunks = b_per_w // C
    assert n_chunks % 2 == 0 and n_chunks >= 4
    mesh = plsc.VectorSubcoreMesh(core_axis_name="c", subcore_axis_name="s")

    @functools.partial(
        pl.kernel,
        mesh=mesh,
        compiler_params=pltpu.CompilerParams(use_tc_tiling_on_sc=False),
        out_type=jax.ShapeDtypeStruct((N, D), jnp.float32),
        scratch_types=[
            pltpu.VMEM((2, C), jnp.int32),
            pltpu.VMEM((2, C, D), jnp.float32),
            pltpu.SemaphoreType.DMA,
            pltpu.SemaphoreType.DMA,
            pltpu.SemaphoreType.DMA,
            pltpu.SemaphoreType.DMA,
            pltpu.SemaphoreType.DMA,
            pltpu.SemaphoreType.DMA,
        ],
    )
    def k(idx_hbm, table_hbm, out_hbm, idx_v, rows_v,
          si0, si1, sg0, sg1, so0, so1):
        wid = lax.axis_index("s") * NC + lax.axis_index("c")
        base = wid * b_per_w
        s_idx = (si0, si1)
        s_g = (sg0, sg1)
        s_out = (so0, so1)

        def idx_copy(i, b):
            return pltpu.make_async_copy(
                idx_hbm.at[pl.ds(base + i * C, C)], idx_v.at[b], s_idx[b])

        def gather(b):
            return pltpu.make_async_copy(
                table_hbm.at[idx_v.at[b]], rows_v.at[b], s_g[b])

        def store(i, b):
            return pltpu.make_async_copy(
                rows_v.at[b], out_hbm.at[pl.ds(base + i * C, C)], s_out[b])

        idx_copy(0, 0).start()
        idx_copy(1, 1).start()

        def pair_body(g, carry):
            for b in (0, 1):
                i = 2 * g + b
                pb = 1 - b

                @pl.when(i >= 1)
                def _():
                    gather(pb).wait()
                    store(i - 1, pb).start()

                    @pl.when(i + 1 < n_chunks)
                    def _():
                        idx_copy(i + 1, pb).start()

                @pl.when(i >= 2)
                def _():
                    store(i - 2, b).wait()

                idx_copy(i, b).wait()
                gather(b).start()
            return carry

        lax.fori_loop(0, n_chunks // 2, pair_body, 0)

        last = n_chunks - 1
        lb = last % 2
        gather(lb).wait()
        store(last, lb).start()
        store(last - 1, 1 - lb).wait()
        store(last, lb).wait()

    return k


@functools.lru_cache(maxsize=None)
def _tc_relayout(B, H, D, Rb):
    grid = B // Rb
    mr = Rb * H * D // 128

    def body(mid_ref, out_ref):
        x = mid_ref[...]
        parts = [x[:, q * D:(q + 1) * D] for q in range(128 // D)]
        y = jnp.stack(parts, axis=1)
        out_ref[...] = y.reshape(Rb, H, D)

    return pl.pallas_call(
        body,
        grid=(grid,),
        in_specs=[pl.BlockSpec((mr, 128), lambda i: (i, 0))],
        out_specs=pl.BlockSpec((Rb, H, D), lambda i: (i, 0, 0)),
        out_shape=jax.ShapeDtypeStruct((B, H, D), jnp.float32),
    )


def kernel(indices, E):
    B, H = indices.shape
    V, D = E.shape
    N = B * H
    idx = indices.reshape(N).astype(jnp.int32)
    mid = _sc_gather(N, D, 1600)(idx, E)
    return mid.reshape(B, H, D)


# speedup vs baseline: 1.0893x; 1.0893x over previous
import functools

import jax
import jax.numpy as jnp
from jax import lax
from jax.experimental import pallas as pl
from jax.experimental.pallas import tpu as pltpu
from jax.experimental.pallas import tpu_sc as plsc

NC, NS = 2, 16
NW = NC * NS


@functools.lru_cache(maxsize=None)
def _sc_gather(N, D, C):
    b_per_w = N // NW
    n_chunks = b_per_w // C
    assert n_chunks % 2 == 0 and n_chunks >= 4
    mesh = plsc.VectorSubcoreMesh(core_axis_name="c", subcore_axis_name="s")

    @functools.partial(
        pl.kernel,
        mesh=mesh,
        compiler_params=pltpu.CompilerParams(use_tc_tiling_on_sc=False),
        out_type=jax.ShapeDtypeStruct((N, D), jnp.float32),
        scratch_types=[
            pltpu.VMEM((2, C), jnp.int32),
            pltpu.VMEM((2, C, D), jnp.float32),
            pltpu.SemaphoreType.DMA,
            pltpu.SemaphoreType.DMA,
            pltpu.SemaphoreType.DMA,
            pltpu.SemaphoreType.DMA,
            pltpu.SemaphoreType.DMA,
            pltpu.SemaphoreType.DMA,
        ],
    )
    def k(idx_hbm, table_hbm, out_hbm, idx_v, rows_v,
          si0, si1, sg0, sg1, so0, so1):
        wid = lax.axis_index("s") * NC + lax.axis_index("c")
        base = wid * b_per_w
        s_idx = (si0, si1)
        s_g = (sg0, sg1)
        s_out = (so0, so1)

        def idx_copy(i, b):
            return pltpu.make_async_copy(
                idx_hbm.at[pl.ds(base + i * C, C)], idx_v.at[b], s_idx[b])

        def gather(b):
            return pltpu.make_async_copy(
                table_hbm.at[idx_v.at[b]], rows_v.at[b], s_g[b])

        def store(i, b):
            return pltpu.make_async_copy(
                rows_v.at[b], out_hbm.at[pl.ds(base + i * C, C)], s_out[b])

        idx_copy(0, 0).start()
        idx_copy(1, 1).start()

        def pair_body(g, carry):
            for b in (0, 1):
                i = 2 * g + b
                pb = 1 - b

                @pl.when(i >= 1)
                def _():
                    gather(pb).wait()
                    store(i - 1, pb).start()

                    @pl.when(i + 1 < n_chunks)
                    def _():
                        idx_copy(i + 1, pb).start()

                @pl.when(i >= 2)
                def _():
                    store(i - 2, b).wait()

                idx_copy(i, b).wait()
                gather(b).start()
            return carry

        lax.fori_loop(0, n_chunks // 2, pair_body, 0)

        last = n_chunks - 1
        lb = last % 2
        gather(lb).wait()
        store(last, lb).start()
        store(last - 1, 1 - lb).wait()
        store(last, lb).wait()

    return k


@functools.lru_cache(maxsize=None)
def _tc_relayout(B, H, D, Rb):
    grid = B // Rb
    mr = Rb * H * D // 128

    def body(mid_ref, out_ref):
        x = mid_ref[...]
        parts = [x[:, q * D:(q + 1) * D] for q in range(128 // D)]
        y = jnp.stack(parts, axis=1)
        out_ref[...] = y.reshape(Rb, H, D)

    return pl.pallas_call(
        body,
        grid=(grid,),
        in_specs=[pl.BlockSpec((mr, 128), lambda i: (i, 0))],
        out_specs=pl.BlockSpec((Rb, H, D), lambda i: (i, 0, 0)),
        out_shape=jax.ShapeDtypeStruct((B, H, D), jnp.float32),
    )


def kernel(indices, E):
    B, H = indices.shape
    V, D = E.shape
    N = B * H
    idx = indices.reshape(N).astype(jnp.int32)
    mid = _sc_gather(N, D, 1600)(idx, E)
    mid128 = mid.reshape(N * D // 128, 128)
    return _tc_relayout(B, H, D, 128)(mid128)
